# Initial kernel scaffold; baseline (speedup 1.0000x reference)
import jax
import jax.numpy as jnp
from jax.experimental import pallas as pl


def kernel(tokens, table):
    raise NotImplementedError("write your pallas kernel here")


# speedup vs baseline: 3.1316x; 3.1316x over previous
import functools

import jax
import jax.numpy as jnp
from jax import lax
from jax.experimental import pallas as pl
from jax.experimental.pallas import tpu as pltpu
from jax.experimental.pallas import tpu_sc as plsc

EMBED_DIM = 128
CHUNK = 128


@functools.lru_cache(maxsize=None)
def _make_lookup(n_idx: int, vocab: int, dim: int):
    info = plsc.get_sparse_core_info()
    nc, ns = info.num_cores, info.num_subcores
    nw = nc * ns
    assert n_idx % (nw * CHUNK) == 0
    n_chunks = n_idx // (nw * CHUNK)
    mesh = plsc.VectorSubcoreMesh(core_axis_name="c", subcore_axis_name="s")

    @functools.partial(
        pl.kernel,
        mesh=mesh,
        out_type=jax.ShapeDtypeStruct((nw, n_chunks, CHUNK, dim), jnp.float32),
        scratch_types=[
            pltpu.VMEM((n_chunks, CHUNK), jnp.int32),
            pltpu.VMEM((CHUNK, dim), jnp.float32),
            pltpu.SemaphoreType.DMA,
        ],
    )
    def lookup(tok_hbm, table_hbm, out_hbm, idx_v, rows_v, sem):
        wid = lax.axis_index("s") * nc + lax.axis_index("c")
        pltpu.sync_copy(tok_hbm.at[wid], idx_v)

        def body(j, carry):
            pltpu.async_copy(table_hbm.at[idx_v.at[j]], rows_v, sem).wait()
            pltpu.sync_copy(rows_v, out_hbm.at[wid, j])
            return carry

        lax.fori_loop(0, n_chunks, body, 0)

    return lookup, nw, n_chunks


def kernel(tokens, table):
    b, t = tokens.shape
    vocab, dim = table.shape
    n_idx = b * t
    lookup, nw, n_chunks = _make_lookup(n_idx, vocab, dim)
    tok = tokens.reshape(nw, n_chunks, CHUNK).astype(jnp.int32)
    out = lookup(tok, table)
    return out.reshape(b, t, dim)


# speedup vs baseline: 3.5863x; 1.1452x over previous
import functools

import jax
import jax.numpy as jnp
from jax import lax
from jax.experimental import pallas as pl
from jax.experimental.pallas import tpu as pltpu
from jax.experimental.pallas import tpu_sc as plsc

EMBED_DIM = 128
CHUNK = 128
NBUF = 4


@functools.lru_cache(maxsize=None)
def _make_lookup(n_idx: int, vocab: int, dim: int):
    info = plsc.get_sparse_core_info()
    nc, ns = info.num_cores, info.num_subcores
    nw = nc * ns
    assert n_idx % (nw * CHUNK) == 0
    n_chunks = n_idx // (nw * CHUNK)
    assert n_chunks % NBUF == 0 and n_chunks >= 2 * NBUF
    mesh = plsc.VectorSubcoreMesh(core_axis_name="c", subcore_axis_name="s")

    @functools.partial(
        pl.kernel,
        mesh=mesh,
        out_type=jax.ShapeDtypeStruct((nw, n_chunks, CHUNK, dim), jnp.float32),
        scratch_types=[
            pltpu.VMEM((n_chunks, CHUNK), jnp.int32),
            pltpu.VMEM((NBUF, CHUNK, dim), jnp.float32),
            pltpu.SemaphoreType.DMA((NBUF,)),
            pltpu.SemaphoreType.DMA((NBUF,)),
        ],
    )
    def lookup(tok_hbm, table_hbm, out_hbm, idx_v, rows_v, sem_in, sem_out):
        wid = lax.axis_index("s") * nc + lax.axis_index("c")
        pltpu.sync_copy(tok_hbm.at[wid], idx_v)

        def fire_gather(g, p):
            pltpu.make_async_copy(
                table_hbm.at[idx_v.at[g]], rows_v.at[p], sem_in.at[p]
            ).start()

        def wait_gather(p):
            pltpu.make_async_copy(
                table_hbm.at[pl.ds(0, CHUNK)], rows_v.at[p], sem_in.at[p]
            ).wait()

        def fire_scatter(g, p):
            pltpu.make_async_copy(
                rows_v.at[p], out_hbm.at[wid, g], sem_out.at[p]
            ).start()

        def wait_scatter(p):
            pltpu.make_async_copy(
                rows_v.at[p], out_hbm.at[wid, 0], sem_out.at[p]
            ).wait()

        fire_gather(0, 0)
        for g in range(1, NBUF):
            fire_gather(g, g)
            wait_gather(g - 1)
            fire_scatter(g - 1, g - 1)

        def body(i, carry):
            g0 = i * NBUF
            for p in range(NBUF):
                g = g0 + p
                wait_scatter(p)
                fire_gather(g, p)
                pm1 = (p - 1) % NBUF
                wait_gather(pm1)
                fire_scatter(g - 1, pm1)
            return carry

        lax.fori_loop(1, n_chunks // NBUF, body, 0)

        wait_gather(NBUF - 1)
        fire_scatter(n_chunks - 1, NBUF - 1)
        for p in range(NBUF):
            wait_scatter(p)

    return lookup, nw, n_chunks


def kernel(tokens, table):
    b, t = tokens.shape
    vocab, dim = table.shape
    n_idx = b * t
    lookup, nw, n_chunks = _make_lookup(n_idx, vocab, dim)
    tok = tokens.reshape(nw, n_chunks, CHUNK).astype(jnp.int32)
    out = lookup(tok, table)
    return out.reshape(b, t, dim)
